# baseline (speedup 1.0000x reference)
import functools

import jax
import jax.numpy as jnp
from jax import lax
from jax.experimental import pallas as pl
from jax.experimental.pallas import tpu as pltpu
from jax.experimental.pallas import tpu_sc as plsc

N = 10000
E = 320000
D = 128
NC = 2
NS = 16
NW = NC * NS
B = 80
K = 4
NB = 128
NSLAB = NB // K
EPT = NB * B
EPAD = EPT * NW
WSCALE = float(1 << 23)
ROW_CHUNK = 632


def _sc_spmm_body(sd_hbm, x_hbm, zeros_hbm, out_hbm,
                  slab0, slab1, dc0, dc1, rows0, rows1, agg_sh,
                  ls0, ls1, gs0, gs1, ss0, ss1):
    slab = (slab0, slab1)
    dstc = (dc0, dc1)
    rows = (rows0, rows1)
    lsem = (ls0, ls1)
    gsem = (gs0, gs1)
    ssem = (ss0, ss1)
    c = lax.axis_index("c")
    s = lax.axis_index("s")
    wid = s * NC + c

    row_off = s * ROW_CHUNK
    last_off = (NS - 1) * ROW_CHUNK
    last_cnt = N - last_off

    @pl.when(s < NS - 1)
    def _zero_main():
        pltpu.sync_copy(zeros_hbm.at[pl.ds(row_off, ROW_CHUNK)],
                        agg_sh.at[pl.ds(row_off, ROW_CHUNK)])

    @pl.when(s == NS - 1)
    def _zero_last():
        pltpu.sync_copy(zeros_hbm.at[pl.ds(last_off, last_cnt)],
                        agg_sh.at[pl.ds(last_off, last_cnt)])

    plsc.subcore_barrier()

    sd_base = wid * (NB * 3 * B)
    SLABW = K * 3 * B

    def _slab_stage(k, m):
        pltpu.async_copy(sd_hbm.at[pl.ds(sd_base + k * SLABW, SLABW)],
                         slab[m], lsem[m])

    def _slab_wait(k, m):
        pltpu.make_async_copy(
            sd_hbm.at[pl.ds(sd_base + k * SLABW, SLABW)], slab[m],
            lsem[m]).wait()

    def _gather(m, t, j):
        pltpu.async_copy(x_hbm.at[slab[m].at[pl.ds(t * 3 * B, B)]],
                         rows[j], gsem[j])

    def _gather_wait(m, t, j):
        pltpu.make_async_copy(x_hbm.at[slab[m].at[pl.ds(t * 3 * B, B)]],
                              rows[j], gsem[j]).wait()

    def _scatter_wait(j):
        pltpu.make_async_copy(rows[j], agg_sh.at[dstc[j]], ssem[j]).wait()

    _slab_stage(0, 0)
    _slab_stage(1, 1)
    _slab_wait(0, 0)
    _gather(0, 0, 0)

    def outer(tt, carry):
        for m in range(2):
            k = tt * 2 + m
            for t in range(K):
                b = k * K + t
                j = t % 2
                j2 = 1 - j

                @pl.when((b >= 1) & (b + 1 < NB))
                def _drain_prev(j2=j2):
                    _scatter_wait(j2)

                if t == K - 1:
                    @pl.when(b + 1 < NB)
                    def _gather_next(m=m, k=k, j2=j2):
                        _slab_wait(k + 1, 1 - m)
                        _gather(1 - m, 0, j2)
                else:
                    @pl.when(b + 1 < NB)
                    def _gather_next(m=m, t=t, j2=j2):
                        _gather(m, t + 1, j2)

                _gather_wait(m, t, j)

                toff = t * 3 * B

                def group_body(g, carry2, m=m, toff=toff, j=j):
                    w16 = (slab[m][pl.ds(toff + 2 * B + g * 16, 16)]
                           .astype(jnp.float32) * (1.0 / WSCALE))
                    dstc[j][pl.ds(g * 16, 16)] = (
                        slab[m][pl.ds(toff + B + g * 16, 16)])
                    for e16 in range(16):
                        wsp = jnp.full((16,), w16[e16], jnp.float32)
                        for f in range(D // 16):
                            rows[j][g * 16 + e16, pl.ds(f * 16, 16)] = (
                                rows[j][g * 16 + e16, pl.ds(f * 16, 16)]
                                * wsp)
                    return carry2

                lax.fori_loop(0, B // 16, group_body, 0)

                pltpu.async_copy(rows[j], agg_sh.at[dstc[j]], ssem[j],
                                 add=True)

            @pl.when(k + 2 < NSLAB)
            def _stage_next(k=k, m=m):
                _slab_stage(k + 2, m)
        return carry

    lax.fori_loop(0, NSLAB // 2, outer, 0)

    for j in range(2):
        _scatter_wait(j)

    plsc.subcore_barrier()

    @pl.when(s < NS - 1)
    def _out_main():
        pltpu.sync_copy(agg_sh.at[pl.ds(row_off, ROW_CHUNK)],
                        out_hbm.at[c, pl.ds(row_off, ROW_CHUNK)])

    @pl.when(s == NS - 1)
    def _out_last():
        pltpu.sync_copy(agg_sh.at[pl.ds(last_off, last_cnt)],
                        out_hbm.at[c, pl.ds(last_off, last_cnt)])


_sc_spmm = functools.partial(
    pl.kernel,
    out_type=jax.ShapeDtypeStruct((NC, N, D), jnp.float32),
    mesh=plsc.VectorSubcoreMesh(core_axis_name="c", subcore_axis_name="s"),
    scratch_types=(
        [pltpu.VMEM((K * 3 * B,), jnp.int32) for _ in range(2)]
        + [pltpu.VMEM((B,), jnp.int32) for _ in range(2)]
        + [pltpu.VMEM((B, D), jnp.float32) for _ in range(2)]
        + [pltpu.VMEM_SHARED((N, D), jnp.float32)]
        + [pltpu.SemaphoreType.DMA for _ in range(6)]
    ),
)(_sc_spmm_body)


def _tc_body(agg_ref, x_ref, w_ref, sw_ref, bias_ref, gamma_ref, beta_ref,
             out_ref):
    a = agg_ref[0] + agg_ref[1]
    pre = jnp.dot(a, w_ref[...], preferred_element_type=jnp.float32)
    pre = pre + jnp.dot(x_ref[...], sw_ref[...],
                        preferred_element_type=jnp.float32)
    pre = pre + bias_ref[...]
    mean = jnp.mean(pre, axis=0, keepdims=True)
    cen = pre - mean
    var = jnp.mean(cen * cen, axis=0, keepdims=True)
    out_ref[...] = cen * lax.rsqrt(var + 1e-5) * gamma_ref[...] + beta_ref[...]


def kernel(x, edge_weight, weight, self_weight, bias, gamma, beta, edge_index):
    pad = EPAD - E
    dst = jnp.concatenate([edge_index[0], jnp.zeros((pad,), jnp.int32)])
    src = jnp.concatenate([edge_index[1], jnp.zeros((pad,), jnp.int32)])
    wi = jnp.concatenate([
        jnp.round(edge_weight * WSCALE).astype(jnp.int32),
        jnp.zeros((pad,), jnp.int32)])
    sd = jnp.stack([src.reshape(NW * NB, B), dst.reshape(NW * NB, B),
                    wi.reshape(NW * NB, B)], axis=1).reshape(-1)
    zeros = jnp.zeros((N, D), jnp.float32)

    agg = _sc_spmm(sd, x, zeros)

    out = pl.pallas_call(
        _tc_body,
        out_shape=jax.ShapeDtypeStruct((N, D), jnp.float32),
    )(agg, x, weight, self_weight,
      bias.reshape(1, D), gamma.reshape(1, D), beta.reshape(1, D))
    return out


# speedup vs baseline: 1.0003x; 1.0003x over previous
import functools

import jax
import jax.numpy as jnp
from jax import lax
from jax.experimental import pallas as pl
from jax.experimental.pallas import tpu as pltpu
from jax.experimental.pallas import tpu_sc as plsc

N = 10000
E = 320000
D = 128
NC = 2
NS = 16
NW = NC * NS
B = 80
K = 4
NB = 128
NSLAB = NB // K
EPT = NB * B
EPAD = EPT * NW
WSCALE = float(1 << 23)
ROW_CHUNK = 632


def _sc_spmm_body(sd_hbm, x_hbm, zeros_hbm, out_hbm,
                  slab0, slab1, sc0, sc1, dc0, dc1, rows0, rows1, agg_sh,
                  ls0, ls1, gs0, gs1, ss0, ss1):
    slab = (slab0, slab1)
    srcb = (sc0, sc1)
    dstc = (dc0, dc1)
    rows = (rows0, rows1)
    lsem = (ls0, ls1)
    gsem = (gs0, gs1)
    ssem = (ss0, ss1)
    c = lax.axis_index("c")
    s = lax.axis_index("s")
    wid = s * NC + c

    row_off = s * ROW_CHUNK
    last_off = (NS - 1) * ROW_CHUNK
    last_cnt = N - last_off

    @pl.when(s < NS - 1)
    def _zero_main():
        pltpu.sync_copy(zeros_hbm.at[pl.ds(row_off, ROW_CHUNK)],
                        agg_sh.at[pl.ds(row_off, ROW_CHUNK)])

    @pl.when(s == NS - 1)
    def _zero_last():
        pltpu.sync_copy(zeros_hbm.at[pl.ds(last_off, last_cnt)],
                        agg_sh.at[pl.ds(last_off, last_cnt)])

    plsc.subcore_barrier()

    sd_base = wid * (NB * 3 * B)
    SLABW = K * 3 * B

    def _slab_stage(k, m):
        pltpu.async_copy(sd_hbm.at[pl.ds(sd_base + k * SLABW, SLABW)],
                         slab[m], lsem[m])

    def _slab_wait(k, m):
        pltpu.make_async_copy(
            sd_hbm.at[pl.ds(sd_base + k * SLABW, SLABW)], slab[m],
            lsem[m]).wait()

    def _src_copy(m, t, j):
        for g in range(B // 16):
            srcb[j][pl.ds(g * 16, 16)] = (
                slab[m][pl.ds(t * 3 * B + g * 16, 16)])

    def _gather(j):
        pltpu.async_copy(x_hbm.at[srcb[j]], rows[j], gsem[j])

    def _gather_wait(j):
        pltpu.make_async_copy(x_hbm.at[srcb[j]], rows[j], gsem[j]).wait()

    def _scatter_wait(j):
        pltpu.make_async_copy(rows[j], agg_sh.at[dstc[j]], ssem[j]).wait()

    _slab_stage(0, 0)
    _slab_stage(1, 1)
    _slab_wait(0, 0)
    _src_copy(0, 0, 0)
    _gather(0)

    def outer(tt, carry):
        for m in range(2):
            k = tt * 2 + m
            for t in range(K):
                b = k * K + t
                j = t % 2
                j2 = 1 - j

                @pl.when((b >= 1) & (b + 1 < NB))
                def _drain_prev(j2=j2):
                    _scatter_wait(j2)

                if t == K - 1:
                    @pl.when(b + 1 < NB)
                    def _gather_next(m=m, k=k, j2=j2):
                        _slab_wait(k + 1, 1 - m)
                        _src_copy(1 - m, 0, j2)
                        _gather(j2)
                else:
                    @pl.when(b + 1 < NB)
                    def _gather_next(m=m, t=t, j2=j2):
                        _src_copy(m, t + 1, j2)
                        _gather(j2)

                _gather_wait(j)

                toff = t * 3 * B

                def group_body(g, carry2, m=m, toff=toff, j=j):
                    w16 = (slab[m][pl.ds(toff + 2 * B + g * 16, 16)]
                           .astype(jnp.float32) * (1.0 / WSCALE))
                    dstc[j][pl.ds(g * 16, 16)] = (
                        slab[m][pl.ds(toff + B + g * 16, 16)])
                    for e16 in range(16):
                        wsp = jnp.full((16,), w16[e16], jnp.float32)
                        for f in range(D // 16):
                            rows[j][g * 16 + e16, pl.ds(f * 16, 16)] = (
                                rows[j][g * 16 + e16, pl.ds(f * 16, 16)]
                                * wsp)
                    return carry2

                lax.fori_loop(0, B // 16, group_body, 0)

                pltpu.async_copy(rows[j], agg_sh.at[dstc[j]], ssem[j],
                                 add=True)

            @pl.when(k + 2 < NSLAB)
            def _stage_next(k=k, m=m):
                _slab_stage(k + 2, m)
        return carry

    lax.fori_loop(0, NSLAB // 2, outer, 0)

    for j in range(2):
        _scatter_wait(j)

    plsc.subcore_barrier()

    @pl.when(s < NS - 1)
    def _out_main():
        pltpu.sync_copy(agg_sh.at[pl.ds(row_off, ROW_CHUNK)],
                        out_hbm.at[c, pl.ds(row_off, ROW_CHUNK)])

    @pl.when(s == NS - 1)
    def _out_last():
        pltpu.sync_copy(agg_sh.at[pl.ds(last_off, last_cnt)],
                        out_hbm.at[c, pl.ds(last_off, last_cnt)])


_sc_spmm = functools.partial(
    pl.kernel,
    out_type=jax.ShapeDtypeStruct((NC, N, D), jnp.float32),
    mesh=plsc.VectorSubcoreMesh(core_axis_name="c", subcore_axis_name="s"),
    scratch_types=(
        [pltpu.VMEM((K * 3 * B,), jnp.int32) for _ in range(2)]
        + [pltpu.VMEM((B,), jnp.int32) for _ in range(4)]
        + [pltpu.VMEM((B, D), jnp.float32) for _ in range(2)]
        + [pltpu.VMEM_SHARED((N, D), jnp.float32)]
        + [pltpu.SemaphoreType.DMA for _ in range(6)]
    ),
)(_sc_spmm_body)


def _tc_body(agg_ref, x_ref, w_ref, sw_ref, bias_ref, gamma_ref, beta_ref,
             out_ref):
    a = agg_ref[0] + agg_ref[1]
    pre = jnp.dot(a, w_ref[...], preferred_element_type=jnp.float32)
    pre = pre + jnp.dot(x_ref[...], sw_ref[...],
                        preferred_element_type=jnp.float32)
    pre = pre + bias_ref[...]
    mean = jnp.mean(pre, axis=0, keepdims=True)
    cen = pre - mean
    var = jnp.mean(cen * cen, axis=0, keepdims=True)
    out_ref[...] = cen * lax.rsqrt(var + 1e-5) * gamma_ref[...] + beta_ref[...]


def kernel(x, edge_weight, weight, self_weight, bias, gamma, beta, edge_index):
    pad = EPAD - E
    dst = jnp.concatenate([edge_index[0], jnp.zeros((pad,), jnp.int32)])
    src = jnp.concatenate([edge_index[1], jnp.zeros((pad,), jnp.int32)])
    wi = jnp.concatenate([
        jnp.round(edge_weight * WSCALE).astype(jnp.int32),
        jnp.zeros((pad,), jnp.int32)])
    sd = jnp.stack([src.reshape(NW * NB, B), dst.reshape(NW * NB, B),
                    wi.reshape(NW * NB, B)], axis=1).reshape(-1)
    zeros = jnp.zeros((N, D), jnp.float32)

    agg = _sc_spmm(sd, x, zeros)

    out = pl.pallas_call(
        _tc_body,
        out_shape=jax.ShapeDtypeStruct((N, D), jnp.float32),
    )(agg, x, weight, self_weight,
      bias.reshape(1, D), gamma.reshape(1, D), beta.reshape(1, D))
    return out
